# Initial kernel scaffold; baseline (speedup 1.0000x reference)
import jax
import jax.numpy as jnp
from jax.experimental import pallas as pl


def kernel(feat0, feat1, feat2, edge_index, e_feat, left, right, mid, params):
    raise NotImplementedError("write your pallas kernel here")


# speedup vs baseline: 1.0750x; 1.0750x over previous
import functools

import jax
import jax.numpy as jnp
from jax.experimental import pallas as pl

N0, N1, N2 = 20000, 15000, 15000
N = N0 + N1 + N2
E = 800000
NT = 3
NUM_ETYPES = 5
HID = 16
NCLS = 16
HEADS = [2, 2, 1]
ALPHA = 0.05
NEG = 0.2
B = 65536
DDIM = NCLS * 4


def _decode_body(le_ref, re_ref, mid_ref, w_ref, out_ref):
    le = le_ref[...]
    re = re_ref[...]
    mid = mid_ref[...]
    acc = jnp.zeros((le.shape[0], 1), jnp.float32)
    for r in range(NUM_ETYPES):
        t = jnp.dot(le, w_ref[r], preferred_element_type=jnp.float32)
        rs = jnp.sum(t * re, axis=1, keepdims=True)
        acc = acc + jnp.where(mid == r, rs, 0.0)
    out_ref[...] = acc


def _decode(le, re, mid, dist_w):
    blk = 2048
    grid = (B // blk,)
    out = pl.pallas_call(
        _decode_body,
        grid=grid,
        in_specs=[
            pl.BlockSpec((blk, DDIM), lambda i: (i, 0)),
            pl.BlockSpec((blk, DDIM), lambda i: (i, 0)),
            pl.BlockSpec((blk, 1), lambda i: (i, 0)),
            pl.BlockSpec((NUM_ETYPES, DDIM, DDIM), lambda i: (0, 0, 0)),
        ],
        out_specs=pl.BlockSpec((blk, 1), lambda i: (i, 0)),
        out_shape=jax.ShapeDtypeStruct((B, 1), jnp.float32),
    )(le, re, mid.reshape(B, 1).astype(jnp.int32), dist_w)
    return out[:, 0]


def _l2(x):
    n = jnp.sqrt(jnp.sum(x * x, axis=1, keepdims=True))
    return x / jnp.maximum(n, 1e-12)


def _aggr(x):
    return x.reshape(x.shape[0], NT, -1).mean(1)


def _layer(h, src, dst, ef, p, l, res_attn, act):
    heads = HEADS[l]
    W = p[f'W{l}']
    out = W.shape[2] // heads
    hs = h.reshape(N, NT, -1)
    feat = jnp.einsum('nti,tio->nto', hs, W).reshape(N, NT, heads, out)
    feat = jnp.transpose(feat, (0, 2, 1, 3)).reshape(N, heads, NT * out)
    el = jnp.sum(feat * p[f'attn_l{l}'][None], axis=-1)
    er = jnp.sum(feat * p[f'attn_r{l}'][None], axis=-1)
    ee = p[f'edge_emb{l}'][ef] @ p[f'attn_e{l}']
    e = el[src] + er[dst] + ee
    e = jnp.where(e > 0, e, NEG * e)
    ex = jnp.exp(e)
    den = jax.ops.segment_sum(ex, dst, num_segments=N)
    a = ex / (den[dst] + 1e-16)
    if res_attn is not None:
        if res_attn.shape[1] != heads:
            res_attn = res_attn.mean(axis=1, keepdims=True)
        a = a * (1.0 - ALPHA) + res_attn * ALPHA
    msg = feat[src] * a[:, :, None]
    rst = jax.ops.segment_sum(msg, dst, num_segments=N)
    if l > 0:
        rv = jnp.einsum('nti,tio->nto', hs, p[f'res_W{l}']).reshape(N, NT, heads, out)
        rv = jnp.transpose(rv, (0, 2, 1, 3)).reshape(N, heads, NT * out)
        rst = rst + rv
    rst = rst + p[f'bias{l}'][None]
    if act:
        rst = jax.nn.elu(rst)
    return rst, a


def kernel(feat0, feat1, feat2, edge_index, e_feat, left, right, mid, params):
    p = params
    src = edge_index[0]
    dst = edge_index[1]
    hs = []
    for i, f in enumerate([feat0, feat1, feat2]):
        nt = f @ p[f'fc_w{i}'] + p[f'fc_b{i}']
        slot = jnp.zeros((f.shape[0], HID * NT), jnp.float32)
        slot = slot.at[:, HID * i:HID * (i + 1)].set(nt)
        hs.append(slot)
    h = jnp.concatenate(hs, axis=0)
    emb = [_aggr(_l2(h))]
    h1, a1 = _layer(h, src, dst, e_feat, p, 0, None, True)
    emb.append(_aggr(_l2(h1.mean(1))))
    h = h1.reshape(N, -1)
    h2, a2 = _layer(h, src, dst, e_feat, p, 1, a1, True)
    emb.append(_aggr(_l2(h2.mean(1))))
    h = h2.reshape(N, -1)
    logits, _ = _layer(h, src, dst, e_feat, p, 2, a2, False)
    logits = logits.mean(1)
    logits = _aggr(_l2(logits))
    emb.append(logits)
    o = jnp.concatenate(emb, axis=1)
    return _decode(o[left], o[right], mid, p['dist_W'])


# speedup vs baseline: 32.1664x; 29.9230x over previous
import functools

import jax
import jax.numpy as jnp
from jax import lax
from jax.experimental import pallas as pl
from jax.experimental.pallas import tpu as pltpu, tpu_sc as plsc

N0, N1, N2 = 20000, 15000, 15000
N = N0 + N1 + N2
E = 800000
NT = 3
NUM_ETYPES = 5
HID = 16
NCLS = 16
HEADS = (2, 2, 1)
ALPHA = 0.05
NEG = 0.2
B = 65536
DDIM = NCLS * 4

NC, NS, L = 2, 16, 16
NW = NC * NS
C = 800
NCH = E // C
KPW = (NCH + NW - 1) // NW
C2 = 320
NCH2 = E // C2
KPW2 = (NCH2 + NW - 1) // NW
RT = N // NS
RZ = 625
BROW = B // NW

_mesh = plsc.VectorSubcoreMesh(core_axis_name="c", subcore_axis_name="s",
                               num_cores=NC, num_subcores=NS)
_sc_params = pltpu.CompilerParams(use_tc_tiling_on_sc=False,
                                  needs_layout_passes=False)


def _i16():
    return lax.iota(jnp.int32, 16)


def _make_pass1(heads):
    ex_shape = (E, 2) if heads == 2 else (E, 1)

    @functools.partial(
        pl.kernel,
        out_type=[jax.ShapeDtypeStruct(ex_shape, jnp.float32),
                  jax.ShapeDtypeStruct((NC, N, 16), jnp.float32)],
        mesh=_mesh,
        compiler_params=_sc_params,
        scratch_types=[
            pltpu.VMEM((C,), jnp.int32),
            pltpu.VMEM((C,), jnp.int32),
            pltpu.VMEM((C,), jnp.int32),
            pltpu.VMEM((C, 16), jnp.float32),
            pltpu.VMEM((C, 16), jnp.float32),
            pltpu.VMEM((C, ex_shape[1]), jnp.float32),
            pltpu.VMEM((C, 16), jnp.float32),
            pltpu.VMEM((16,), jnp.float32),
            pltpu.VMEM_SHARED((N, 16), jnp.float32),
        ],
    )
    def pass1(src_h, dst_h, ef_h, el_h, er_h, eet_h, zden_h, zex_h,
              ex_h, denp_h,
              src_v, dst_v, ef_v, els_v, erd_v, exb_v, exw_v, eet_v, den_sh):
        c = lax.axis_index("c")
        s = lax.axis_index("s")
        wid = s * NC + c
        for z in range(RT // RZ):
            pltpu.sync_copy(zden_h, den_sh.at[pl.ds(s * RT + z * RZ, RZ)])
        pltpu.sync_copy(zex_h, exw_v)
        pltpu.sync_copy(eet_h, eet_v)
        plsc.subcore_barrier()

        i16 = _i16()
        r8 = i16 // 2
        pc = i16 & 1
        z16 = jnp.zeros((16,), jnp.int32)

        def chunk(k, carry):
            ch = wid + k * NW

            @pl.when(ch < NCH)
            def _():
                off = ch * C
                pltpu.sync_copy(src_h.at[pl.ds(off, C)], src_v)
                pltpu.sync_copy(dst_h.at[pl.ds(off, C)], dst_v)
                pltpu.sync_copy(ef_h.at[pl.ds(off, C)], ef_v)
                pltpu.sync_copy(el_h.at[src_v], els_v)
                pltpu.sync_copy(er_h.at[dst_v], erd_v)

                if heads == 2:
                    def grp(g, cy):
                        rows = r8 + g * 8
                        elv = plsc.load_gather(els_v, [rows, pc])
                        erv = plsc.load_gather(erd_v, [rows, pc])
                        efr = plsc.load_gather(ef_v, [rows])
                        ee = plsc.load_gather(eet_v, [efr * 2 + pc])
                        e = elv + erv + ee
                        e = jnp.where(e > 0, e, NEG * e)
                        ex = jnp.exp(e)
                        plsc.store_scatter(exb_v, [rows, pc], ex)
                        plsc.store_scatter(exw_v, [rows, pc], ex)
                        return cy
                    lax.fori_loop(0, C // 8, grp, 0)
                else:
                    def grp(g, cy):
                        rows = i16 + g * 16
                        elv = plsc.load_gather(els_v, [rows, z16])
                        erv = plsc.load_gather(erd_v, [rows, z16])
                        efr = plsc.load_gather(ef_v, [rows])
                        ee = plsc.load_gather(eet_v, [efr])
                        e = elv + erv + ee
                        e = jnp.where(e > 0, e, NEG * e)
                        ex = jnp.exp(e)
                        plsc.store_scatter(exb_v, [rows, z16], ex)
                        plsc.store_scatter(exw_v, [rows, z16], ex)
                        return cy
                    lax.fori_loop(0, C // 16, grp, 0)

                pltpu.sync_copy(exb_v, ex_h.at[pl.ds(off, C)])
                pltpu.sync_copy(exw_v, den_sh.at[dst_v], add=True)
            return carry

        lax.fori_loop(0, KPW, chunk, 0)
        plsc.subcore_barrier()
        pltpu.sync_copy(den_sh.at[pl.ds(s * RT, RT)],
                        denp_h.at[c, pl.ds(s * RT, RT)])

    return pass1


def _make_passa(heads, hp):
    h2 = heads == 2
    a_shape = (E, 2) if h2 else (E, 1)
    scratch = [
        pltpu.VMEM((C,), jnp.int32),
        pltpu.VMEM((C, 2) if h2 else (C, 1), jnp.float32),
        pltpu.VMEM((C, 16), jnp.float32),
        pltpu.VMEM((C, 16), jnp.float32),
        pltpu.VMEM((C, 2) if h2 else (C, 1), jnp.float32),
    ]
    if hp is not None:
        scratch.append(pltpu.VMEM((C, 2) if hp == 2 else (C, 1), jnp.float32))

    @functools.partial(
        pl.kernel,
        out_type=jax.ShapeDtypeStruct(a_shape, jnp.float32),
        mesh=_mesh,
        compiler_params=_sc_params,
        scratch_types=scratch,
    )
    def passa(*refs):
        if hp is not None:
            (dst_h, ex_h, d0_h, d1_h, ap_h, a_h,
             dst_v, exb_v, d0_v, d1_v, ab_v, apb_v) = refs
        else:
            (dst_h, ex_h, d0_h, d1_h, a_h,
             dst_v, exb_v, d0_v, d1_v, ab_v) = refs
            apb_v = None
        c = lax.axis_index("c")
        s = lax.axis_index("s")
        wid = s * NC + c
        i16 = _i16()
        r8 = i16 // 2
        pc = i16 & 1
        z16 = jnp.zeros((16,), jnp.int32)

        def chunk(k, carry):
            ch = wid + k * NW

            @pl.when(ch < NCH)
            def _():
                off = ch * C
                pltpu.sync_copy(dst_h.at[pl.ds(off, C)], dst_v)
                pltpu.sync_copy(ex_h.at[pl.ds(off, C)], exb_v)
                pltpu.sync_copy(d0_h.at[dst_v], d0_v)
                pltpu.sync_copy(d1_h.at[dst_v], d1_v)
                if apb_v is not None:
                    pltpu.sync_copy(ap_h.at[pl.ds(off, C)], apb_v)

                if h2:
                    def grp(g, cy):
                        rows = r8 + g * 8
                        ex = plsc.load_gather(exb_v, [rows, pc])
                        d0 = plsc.load_gather(d0_v, [rows, pc])
                        d1 = plsc.load_gather(d1_v, [rows, pc])
                        a = ex / (d0 + d1 + 1e-16)
                        if apb_v is not None:
                            ap = plsc.load_gather(apb_v, [rows, pc])
                            a = a * (1.0 - ALPHA) + ap * ALPHA
                        plsc.store_scatter(ab_v, [rows, pc], a)
                        return cy
                    lax.fori_loop(0, C // 8, grp, 0)
                else:
                    def grp(g, cy):
                        rows = i16 + g * 16
                        ex = plsc.load_gather(exb_v, [rows, z16])
                        d0 = plsc.load_gather(d0_v, [rows, z16])
                        d1 = plsc.load_gather(d1_v, [rows, z16])
                        a = ex / (d0 + d1 + 1e-16)
                        if apb_v is not None:
                            ap0 = plsc.load_gather(apb_v, [rows, z16])
                            ap1 = plsc.load_gather(apb_v, [rows, z16 + 1])
                            a = a * (1.0 - ALPHA) + (ap0 + ap1) * (0.5 * ALPHA)
                        plsc.store_scatter(ab_v, [rows, z16], a)
                        return cy
                    lax.fori_loop(0, C // 16, grp, 0)

                pltpu.sync_copy(ab_v, a_h.at[pl.ds(off, C)])
            return carry

        lax.fori_loop(0, KPW, chunk, 0)

    return passa


def _make_pass2(heads, W, hmap):
    h2 = heads == 2
    nv = W // 16
    assert len(hmap) == nv

    @functools.partial(
        pl.kernel,
        out_type=jax.ShapeDtypeStruct((NC, N, W), jnp.float32),
        mesh=_mesh,
        compiler_params=_sc_params,
        scratch_types=[
            pltpu.VMEM((C2,), jnp.int32),
            pltpu.VMEM((C2,), jnp.int32),
            pltpu.VMEM((C2, 2) if h2 else (C2, 1), jnp.float32),
            pltpu.VMEM((C2, W), jnp.float32),
            pltpu.VMEM_SHARED((N, W), jnp.float32),
        ],
    )
    def pass2(src_h, dst_h, a_h, ftab_h, zero_h, rstp_h,
              src_v, dst_v, ab_v, frows_v, acc_sh):
        c = lax.axis_index("c")
        s = lax.axis_index("s")
        wid = s * NC + c
        for z in range(RT // RZ):
            pltpu.sync_copy(zero_h, acc_sh.at[pl.ds(s * RT + z * RZ, RZ)])
        plsc.subcore_barrier()

        i16 = _i16()
        z16 = jnp.zeros((16,), jnp.int32)
        cols = [i16 + 16 * t for t in range(nv)]

        def chunk(k, carry):
            ch = wid + k * NW

            @pl.when(ch < NCH2)
            def _():
                off = ch * C2
                pltpu.sync_copy(src_h.at[pl.ds(off, C2)], src_v)
                pltpu.sync_copy(dst_h.at[pl.ds(off, C2)], dst_v)
                pltpu.sync_copy(a_h.at[pl.ds(off, C2)], ab_v)
                pltpu.sync_copy(ftab_h.at[src_v], frows_v)

                ng = C2 // 8 if h2 else C2 // 16
                epg = 8 if h2 else 16

                def grp(g, cy):
                    for j in range(epg):
                        eloc = g * epg + j
                        rowv = z16 + eloc
                        if h2:
                            bs = {}
                            for h in set(hmap):
                                bs[h] = plsc.load_gather(ab_v, [rowv, z16 + h])
                        else:
                            b = plsc.load_gather(ab_v, [rowv, z16])
                            bs = {h: b for h in set(hmap)}
                        for t in range(nv):
                            r = plsc.load_gather(frows_v, [rowv, cols[t]])
                            plsc.store_scatter(frows_v, [rowv, cols[t]],
                                               r * bs[hmap[t]])
                    return cy

                lax.fori_loop(0, ng, grp, 0)
                pltpu.sync_copy(frows_v, acc_sh.at[dst_v], add=True)
            return carry

        lax.fori_loop(0, KPW2, chunk, 0)
        plsc.subcore_barrier()
        pltpu.sync_copy(acc_sh.at[pl.ds(s * RT, RT)],
                        rstp_h.at[c, pl.ds(s * RT, RT)])

    return pass2


_pass1_h2 = _make_pass1(2)
_pass1_h1 = _make_pass1(1)
_passa_l0 = _make_passa(2, None)
_passa_l1 = _make_passa(2, 2)
_passa_l2 = _make_passa(1, 2)
_pass2_h2_00 = _make_pass2(2, 32, (0, 0))
_pass2_h2_01 = _make_pass2(2, 32, (0, 1))
_pass2_h2_11 = _make_pass2(2, 32, (1, 1))
_pass2_h1_32 = _make_pass2(1, 32, (0, 0))
_pass2_h1_16 = _make_pass2(1, 16, (0,))


@functools.partial(
    pl.kernel,
    out_type=[jax.ShapeDtypeStruct((B, DDIM), jnp.float32),
              jax.ShapeDtypeStruct((B, DDIM), jnp.float32)],
    mesh=_mesh,
    compiler_params=_sc_params,
    scratch_types=[
        pltpu.VMEM((1024,), jnp.int32),
        pltpu.VMEM((1024, DDIM), jnp.float32),
    ],
)
def _sc_decode_gather(o_h, left_h, right_h, le_h, re_h, idx_v, rows_v):
    c = lax.axis_index("c")
    s = lax.axis_index("s")
    wid = s * NC + c
    base = wid * BROW

    def chunk(k, carry):
        off = base + k * 1024
        pltpu.sync_copy(left_h.at[pl.ds(off, 1024)], idx_v)
        pltpu.sync_copy(o_h.at[idx_v], rows_v)
        pltpu.sync_copy(rows_v, le_h.at[pl.ds(off, 1024)])
        pltpu.sync_copy(right_h.at[pl.ds(off, 1024)], idx_v)
        pltpu.sync_copy(o_h.at[idx_v], rows_v)
        pltpu.sync_copy(rows_v, re_h.at[pl.ds(off, 1024)])
        return carry

    lax.fori_loop(0, BROW // 1024, chunk, 0)


BLK = 1000
NB = N // BLK


def _tmap(i):
    return (i >= N0 // BLK).astype(jnp.int32) + (i >= (N0 + N1) // BLK).astype(jnp.int32)


def _l2n(x):
    return x / jnp.maximum(jnp.sqrt(jnp.sum(x * x, axis=1, keepdims=True)), 1e-12)


def _slotmean(x):
    return (x[:, :16] + x[:, 16:32] + x[:, 32:48]) * (1.0 / 3.0)


def _elu(x):
    return jnp.where(x > 0, x, jnp.exp(jnp.minimum(x, 0.0)) - 1.0)


def _heads_el(ff, av, heads, d):
    parts = [jnp.sum(ff[:, h * d:(h + 1) * d] * av[0, h * d:(h + 1) * d][None],
                     axis=1, keepdims=True) for h in range(heads)]
    parts.append(jnp.zeros((ff.shape[0], 16 - heads), jnp.float32))
    return jnp.concatenate(parts, axis=1)


def _k0_body(f_ref, fcw_ref, fcb_ref, wb0_ref, al_ref, ar_ref, eeb_ref, aeb_ref,
             fa_ref, fb_ref, fc_ref, el_ref, er_ref, emb_ref, eet_ref):
    nt = jnp.dot(f_ref[...], fcw_ref[0], preferred_element_type=jnp.float32)
    nt = nt + fcb_ref[0, 0][None]
    ff = jnp.dot(nt, wb0_ref[0], preferred_element_type=jnp.float32)
    fa_ref[...] = ff[:, :32]
    fb_ref[...] = ff[:, 32:64]
    fc_ref[...] = ff[:, 64:96]
    el_ref[...] = _heads_el(ff, al_ref[...], 2, 48)
    er_ref[...] = _heads_el(ff, ar_ref[...], 2, 48)
    nn = jnp.maximum(jnp.sqrt(jnp.sum(nt * nt, axis=1, keepdims=True)), 1e-12)
    emb_ref[...] = nt / (3.0 * nn)
    eet_ref[...] = jnp.dot(eeb_ref[...], aeb_ref[...],
                           preferred_element_type=jnp.float32)


def _k12_body(pa_ref, pb_ref, pc_ref, rv_ref, bias_ref, wb_ref, rwb_ref,
              al_ref, ar_ref,
              emb_ref, fa_ref, fb_ref, fc_ref, el_ref, er_ref, rvo_ref,
              *, heads_out, d_out):
    rst = jnp.concatenate([pa_ref[0] + pa_ref[1], pb_ref[0] + pb_ref[1],
                           pc_ref[0] + pc_ref[1]], axis=1)
    if rv_ref is not None:
        rst = rst + rv_ref[...]
    rst = rst + bias_ref[...]
    h = _elu(rst)
    hm = (h[:, :48] + h[:, 48:]) * 0.5
    emb_ref[...] = _slotmean(_l2n(hm))
    ff = jnp.dot(h, wb_ref[...], preferred_element_type=jnp.float32)
    rvo_ref[...] = jnp.dot(h, rwb_ref[...], preferred_element_type=jnp.float32)
    fa_ref[...] = ff[:, :32]
    if d_out == 96:
        fb_ref[...] = ff[:, 32:64]
        fc_ref[...] = ff[:, 64:96]
    else:
        fb_ref[...] = ff[:, 32:48]
    el_ref[...] = _heads_el(ff, al_ref[...], heads_out, d_out // heads_out)
    er_ref[...] = _heads_el(ff, ar_ref[...], heads_out, d_out // heads_out)


def _k3_body(pa_ref, pb_ref, rv_ref, bias_ref, e0_ref, e1_ref, e2_ref, o_ref):
    logits = jnp.concatenate([pa_ref[0] + pa_ref[1], pb_ref[0] + pb_ref[1]],
                             axis=1)
    logits = logits + rv_ref[...] + bias_ref[...]
    emb3 = _slotmean(_l2n(logits))
    o_ref[...] = jnp.concatenate([e0_ref[...], e1_ref[...], e2_ref[...], emb3],
                                 axis=1)


def _decode_body(le_ref, re_ref, mid_ref, w_ref, out_ref):
    le = le_ref[...]
    re = re_ref[...]
    mid = mid_ref[...]
    acc = jnp.zeros((le.shape[0], 1), jnp.float32)
    for r in range(NUM_ETYPES):
        t = jnp.dot(le, w_ref[r], preferred_element_type=jnp.float32)
        rs = jnp.sum(t * re, axis=1, keepdims=True)
        acc = acc + jnp.where(mid == r, rs, 0.0)
    out_ref[...] = acc


def _rowspec(w):
    return pl.BlockSpec((BLK, w), lambda i: (i, 0))


def _pspec(w):
    return pl.BlockSpec((2, BLK, w), lambda i: (0, i, 0))


def _whole(shape):
    nd = len(shape)
    return pl.BlockSpec(shape, lambda i: (0,) * nd)


def kernel(feat0, feat1, feat2, edge_index, e_feat, left, right, mid, params):
    p = params
    src = edge_index[0].astype(jnp.int32)
    dst = edge_index[1].astype(jnp.int32)
    ef = e_feat.astype(jnp.int32)

    featcat = jnp.concatenate([feat0, feat1, feat2], axis=0)
    fcw = jnp.stack([p['fc_w0'], p['fc_w1'], p['fc_w2']])
    fcb = jnp.stack([p['fc_b0'], p['fc_b1'], p['fc_b2']]).reshape(3, 1, 16)

    def blockdiag(W, heads, out):
        Wr = W.reshape(NT, W.shape[1], heads, out)
        Wb = jnp.zeros((NT * W.shape[1], heads * NT * out), jnp.float32)
        for t in range(NT):
            for h in range(heads):
                Wb = Wb.at[t * W.shape[1]:(t + 1) * W.shape[1],
                           h * NT * out + t * out:h * NT * out + (t + 1) * out
                           ].set(Wr[t, :, h, :])
        return Wb

    wb0s = blockdiag(p['W0'], 2, 16).reshape(NT, 16, 96)
    wb1 = blockdiag(p['W1'], 2, 16)
    rwb1 = blockdiag(p['res_W1'], 2, 16)
    wb2 = blockdiag(p['W2'], 1, 16)
    rwb2 = blockdiag(p['res_W2'], 1, 16)
    al = [p[f'attn_l{l}'].reshape(1, -1) for l in range(3)]
    ar = [p[f'attn_r{l}'].reshape(1, -1) for l in range(3)]
    bias = [p[f'bias{l}'].reshape(1, -1) for l in range(3)]
    eeb = jnp.zeros((16, 48), jnp.float32)
    aeb = jnp.zeros((48, 128), jnp.float32)
    for l in range(3):
        eeb = eeb.at[5 * l:5 * l + 5, 16 * l:16 * l + 16].set(p[f'edge_emb{l}'])
        aeb = aeb.at[16 * l:16 * l + 16, 2 * l:2 * l + HEADS[l]].set(p[f'attn_e{l}'])
    zden = jnp.zeros((RZ, 16), jnp.float32)
    zex = jnp.zeros((C, 16), jnp.float32)
    zero32 = jnp.zeros((RZ, 32), jnp.float32)
    zero16 = jnp.zeros((RZ, 16), jnp.float32)

    wmap = lambda i: (_tmap(i), 0, 0)
    fA0, fB0, fC0, el0, er0, emb0, eetabs = pl.pallas_call(
        _k0_body,
        grid=(NB,),
        in_specs=[
            _rowspec(128),
            pl.BlockSpec((1, 128, 16), wmap),
            pl.BlockSpec((1, 1, 16), wmap),
            pl.BlockSpec((1, 16, 96), wmap),
            _whole((1, 96)), _whole((1, 96)),
            _whole((16, 48)), _whole((48, 128)),
        ],
        out_specs=[_rowspec(32), _rowspec(32), _rowspec(32),
                   _rowspec(16), _rowspec(16), _rowspec(16),
                   _whole((16, 128))],
        out_shape=[jax.ShapeDtypeStruct((N, 32), jnp.float32),
                   jax.ShapeDtypeStruct((N, 32), jnp.float32),
                   jax.ShapeDtypeStruct((N, 32), jnp.float32),
                   jax.ShapeDtypeStruct((N, 16), jnp.float32),
                   jax.ShapeDtypeStruct((N, 16), jnp.float32),
                   jax.ShapeDtypeStruct((N, 16), jnp.float32),
                   jax.ShapeDtypeStruct((16, 128), jnp.float32)],
    )(featcat, fcw, fcb, wb0s, al[0], ar[0], eeb, aeb)

    def eet_flat(l):
        t = eetabs[5 * l:5 * l + 5, 2 * l:2 * l + HEADS[l]]
        return jnp.zeros((16,), jnp.float32).at[:5 * HEADS[l]].set(t.reshape(-1))

    ex0, denp0 = _pass1_h2(src, dst, ef, el0, er0, eet_flat(0), zden, zex)
    a0 = _passa_l0(dst, ex0, denp0[0], denp0[1])
    pA0 = _pass2_h2_00(src, dst, a0, fA0, zero32)
    pB0 = _pass2_h2_01(src, dst, a0, fB0, zero32)
    pC0 = _pass2_h2_11(src, dst, a0, fC0, zero32)

    k1 = functools.partial(_k12_body, heads_out=2, d_out=96)

    def k1_body(pa, pb, pc, bias_r, wb_r, rwb_r, al_r, ar_r,
                emb_r, fa_r, fb_r, fc_r, el_r, er_r, rvo_r):
        k1(pa, pb, pc, None, bias_r, wb_r, rwb_r, al_r, ar_r,
           emb_r, fa_r, fb_r, fc_r, el_r, er_r, rvo_r)

    emb1, fA1, fB1, fC1, el1, er1, rv1 = pl.pallas_call(
        k1_body,
        grid=(NB,),
        in_specs=[_pspec(32), _pspec(32), _pspec(32),
                  _whole((1, 96)), _whole((96, 96)), _whole((96, 96)),
                  _whole((1, 96)), _whole((1, 96))],
        out_specs=[_rowspec(16), _rowspec(32), _rowspec(32), _rowspec(32),
                   _rowspec(16), _rowspec(16), _rowspec(96)],
        out_shape=[jax.ShapeDtypeStruct((N, 16), jnp.float32),
                   jax.ShapeDtypeStruct((N, 32), jnp.float32),
                   jax.ShapeDtypeStruct((N, 32), jnp.float32),
                   jax.ShapeDtypeStruct((N, 32), jnp.float32),
                   jax.ShapeDtypeStruct((N, 16), jnp.float32),
                   jax.ShapeDtypeStruct((N, 16), jnp.float32),
                   jax.ShapeDtypeStruct((N, 96), jnp.float32)],
    )(pA0, pB0, pC0, bias[0], wb1, rwb1, al[1], ar[1])

    ex1, denp1 = _pass1_h2(src, dst, ef, el1, er1, eet_flat(1), zden, zex)
    a1 = _passa_l1(dst, ex1, denp1[0], denp1[1], a0)
    pA1 = _pass2_h2_00(src, dst, a1, fA1, zero32)
    pB1 = _pass2_h2_01(src, dst, a1, fB1, zero32)
    pC1 = _pass2_h2_11(src, dst, a1, fC1, zero32)

    k2 = functools.partial(_k12_body, heads_out=1, d_out=48)

    def k2_body(pa, pb, pc, rv_r, bias_r, wb_r, rwb_r, al_r, ar_r,
                emb_r, fa_r, fb_r, el_r, er_r, rvo_r):
        k2(pa, pb, pc, rv_r, bias_r, wb_r, rwb_r, al_r, ar_r,
           emb_r, fa_r, fb_r, None, el_r, er_r, rvo_r)

    emb2, fA2, fB2, el2, er2, rv2 = pl.pallas_call(
        k2_body,
        grid=(NB,),
        in_specs=[_pspec(32), _pspec(32), _pspec(32), _rowspec(96),
                  _whole((1, 96)), _whole((96, 48)), _whole((96, 48)),
                  _whole((1, 48)), _whole((1, 48))],
        out_specs=[_rowspec(16), _rowspec(32), _rowspec(16),
                   _rowspec(16), _rowspec(16), _rowspec(48)],
        out_shape=[jax.ShapeDtypeStruct((N, 16), jnp.float32),
                   jax.ShapeDtypeStruct((N, 32), jnp.float32),
                   jax.ShapeDtypeStruct((N, 16), jnp.float32),
                   jax.ShapeDtypeStruct((N, 16), jnp.float32),
                   jax.ShapeDtypeStruct((N, 16), jnp.float32),
                   jax.ShapeDtypeStruct((N, 48), jnp.float32)],
    )(pA1, pB1, pC1, rv1, bias[1], wb2, rwb2, al[2], ar[2])

    ex2, denp2 = _pass1_h1(src, dst, ef, el2, er2, eet_flat(2), zden, zex)
    a2 = _passa_l2(dst, ex2, denp2[0], denp2[1], a1)
    pA2 = _pass2_h1_32(src, dst, a2, fA2, zero32)
    pB2 = _pass2_h1_16(src, dst, a2, fB2, zero16)

    o = pl.pallas_call(
        _k3_body,
        grid=(NB,),
        in_specs=[_pspec(32), _pspec(16), _rowspec(48), _whole((1, 48)),
                  _rowspec(16), _rowspec(16), _rowspec(16)],
        out_specs=_rowspec(64),
        out_shape=jax.ShapeDtypeStruct((N, 64), jnp.float32),
    )(pA2, pB2, rv2, bias[2], emb0, emb1, emb2)

    le, re = _sc_decode_gather(o, left.astype(jnp.int32), right.astype(jnp.int32))
    blk = 2048
    out = pl.pallas_call(
        _decode_body,
        grid=(B // blk,),
        in_specs=[
            pl.BlockSpec((blk, DDIM), lambda i: (i, 0)),
            pl.BlockSpec((blk, DDIM), lambda i: (i, 0)),
            pl.BlockSpec((blk, 1), lambda i: (i, 0)),
            pl.BlockSpec((NUM_ETYPES, DDIM, DDIM), lambda i: (0, 0, 0)),
        ],
        out_specs=pl.BlockSpec((blk, 1), lambda i: (i, 0)),
        out_shape=jax.ShapeDtypeStruct((B, 1), jnp.float32),
    )(le, re, mid.reshape(B, 1).astype(jnp.int32), p['dist_W'])
    return out[:, 0]


# speedup vs baseline: 37.0200x; 1.1509x over previous
import functools

import jax
import jax.numpy as jnp
from jax import lax
from jax.experimental import pallas as pl
from jax.experimental.pallas import tpu as pltpu, tpu_sc as plsc

N0, N1, N2 = 20000, 15000, 15000
N = N0 + N1 + N2
E = 800000
NT = 3
NUM_ETYPES = 5
HID = 16
NCLS = 16
HEADS = (2, 2, 1)
ALPHA = 0.05
NEG = 0.2
B = 65536
DDIM = NCLS * 4

NC, NS, L = 2, 16, 16
NW = NC * NS
C = 800
NCH = E // C
KPW = (NCH + NW - 1) // NW
C2 = 256
NCH2 = E // C2
KPW2 = (NCH2 + NW - 1) // NW
RT = N // NS
RZ = 625
BROW = B // NW

_mesh = plsc.VectorSubcoreMesh(core_axis_name="c", subcore_axis_name="s",
                               num_cores=NC, num_subcores=NS)
_sc_params = pltpu.CompilerParams(use_tc_tiling_on_sc=False,
                                  needs_layout_passes=False)


def _i16():
    return lax.iota(jnp.int32, 16)


def _make_pass1(heads):
    ex_shape = (E, 2) if heads == 2 else (E, 1)

    @functools.partial(
        pl.kernel,
        out_type=[jax.ShapeDtypeStruct(ex_shape, jnp.float32),
                  jax.ShapeDtypeStruct((NC, N, 16), jnp.float32)],
        mesh=_mesh,
        compiler_params=_sc_params,
        scratch_types=[
            pltpu.VMEM((C,), jnp.int32),
            pltpu.VMEM((C,), jnp.int32),
            pltpu.VMEM((C,), jnp.int32),
            pltpu.VMEM((C, 16), jnp.float32),
            pltpu.VMEM((C, 16), jnp.float32),
            pltpu.VMEM((C, ex_shape[1]), jnp.float32),
            pltpu.VMEM((C, 16), jnp.float32),
            pltpu.VMEM((16,), jnp.float32),
            pltpu.VMEM_SHARED((N, 16), jnp.float32),
        ],
    )
    def pass1(src_h, dst_h, ef_h, el_h, er_h, eet_h, zden_h, zex_h,
              ex_h, denp_h,
              src_v, dst_v, ef_v, els_v, erd_v, exb_v, exw_v, eet_v, den_sh):
        c = lax.axis_index("c")
        s = lax.axis_index("s")
        wid = s * NC + c
        for z in range(RT // RZ):
            pltpu.sync_copy(zden_h, den_sh.at[pl.ds(s * RT + z * RZ, RZ)])
        pltpu.sync_copy(zex_h, exw_v)
        pltpu.sync_copy(eet_h, eet_v)
        plsc.subcore_barrier()

        i16 = _i16()
        r8 = i16 // 2
        pc = i16 & 1
        z16 = jnp.zeros((16,), jnp.int32)

        def chunk(k, carry):
            ch = wid + k * NW

            @pl.when(ch < NCH)
            def _():
                off = ch * C
                pltpu.sync_copy(src_h.at[pl.ds(off, C)], src_v)
                pltpu.sync_copy(dst_h.at[pl.ds(off, C)], dst_v)
                pltpu.sync_copy(ef_h.at[pl.ds(off, C)], ef_v)
                pltpu.sync_copy(el_h.at[src_v], els_v)
                pltpu.sync_copy(er_h.at[dst_v], erd_v)

                if heads == 2:
                    def grp(g, cy):
                        rows = r8 + g * 8
                        elv = plsc.load_gather(els_v, [rows, pc])
                        erv = plsc.load_gather(erd_v, [rows, pc])
                        efr = plsc.load_gather(ef_v, [rows])
                        ee = plsc.load_gather(eet_v, [efr * 2 + pc])
                        e = elv + erv + ee
                        e = jnp.where(e > 0, e, NEG * e)
                        ex = jnp.exp(e)
                        plsc.store_scatter(exb_v, [rows, pc], ex)
                        plsc.store_scatter(exw_v, [rows, pc], ex)
                        return cy
                    lax.fori_loop(0, C // 8, grp, 0)
                else:
                    def grp(g, cy):
                        rows = i16 + g * 16
                        elv = plsc.load_gather(els_v, [rows, z16])
                        erv = plsc.load_gather(erd_v, [rows, z16])
                        efr = plsc.load_gather(ef_v, [rows])
                        ee = plsc.load_gather(eet_v, [efr])
                        e = elv + erv + ee
                        e = jnp.where(e > 0, e, NEG * e)
                        ex = jnp.exp(e)
                        plsc.store_scatter(exb_v, [rows, z16], ex)
                        plsc.store_scatter(exw_v, [rows, z16], ex)
                        return cy
                    lax.fori_loop(0, C // 16, grp, 0)

                pltpu.sync_copy(exb_v, ex_h.at[pl.ds(off, C)])
                pltpu.sync_copy(exw_v, den_sh.at[dst_v], add=True)
            return carry

        lax.fori_loop(0, KPW, chunk, 0)
        plsc.subcore_barrier()
        pltpu.sync_copy(den_sh.at[pl.ds(s * RT, RT)],
                        denp_h.at[c, pl.ds(s * RT, RT)])

    return pass1


def _make_passa(heads, hp):
    h2 = heads == 2
    a_shape = (E, 2) if h2 else (E, 1)
    scratch = [
        pltpu.VMEM((C,), jnp.int32),
        pltpu.VMEM((C, 2) if h2 else (C, 1), jnp.float32),
        pltpu.VMEM((C, 16), jnp.float32),
        pltpu.VMEM((C, 16), jnp.float32),
        pltpu.VMEM((C, 2) if h2 else (C, 1), jnp.float32),
    ]
    if hp is not None:
        scratch.append(pltpu.VMEM((C, 2) if hp == 2 else (C, 1), jnp.float32))

    @functools.partial(
        pl.kernel,
        out_type=jax.ShapeDtypeStruct(a_shape, jnp.float32),
        mesh=_mesh,
        compiler_params=_sc_params,
        scratch_types=scratch,
    )
    def passa(*refs):
        if hp is not None:
            (dst_h, ex_h, d0_h, d1_h, ap_h, a_h,
             dst_v, exb_v, d0_v, d1_v, ab_v, apb_v) = refs
        else:
            (dst_h, ex_h, d0_h, d1_h, a_h,
             dst_v, exb_v, d0_v, d1_v, ab_v) = refs
            apb_v = None
        c = lax.axis_index("c")
        s = lax.axis_index("s")
        wid = s * NC + c
        i16 = _i16()
        r8 = i16 // 2
        pc = i16 & 1
        z16 = jnp.zeros((16,), jnp.int32)

        def chunk(k, carry):
            ch = wid + k * NW

            @pl.when(ch < NCH)
            def _():
                off = ch * C
                pltpu.sync_copy(dst_h.at[pl.ds(off, C)], dst_v)
                pltpu.sync_copy(ex_h.at[pl.ds(off, C)], exb_v)
                pltpu.sync_copy(d0_h.at[dst_v], d0_v)
                pltpu.sync_copy(d1_h.at[dst_v], d1_v)
                if apb_v is not None:
                    pltpu.sync_copy(ap_h.at[pl.ds(off, C)], apb_v)

                if h2:
                    def grp(g, cy):
                        rows = r8 + g * 8
                        ex = plsc.load_gather(exb_v, [rows, pc])
                        d0 = plsc.load_gather(d0_v, [rows, pc])
                        d1 = plsc.load_gather(d1_v, [rows, pc])
                        a = ex / (d0 + d1 + 1e-16)
                        if apb_v is not None:
                            ap = plsc.load_gather(apb_v, [rows, pc])
                            a = a * (1.0 - ALPHA) + ap * ALPHA
                        plsc.store_scatter(ab_v, [rows, pc], a)
                        return cy
                    lax.fori_loop(0, C // 8, grp, 0)
                else:
                    def grp(g, cy):
                        rows = i16 + g * 16
                        ex = plsc.load_gather(exb_v, [rows, z16])
                        d0 = plsc.load_gather(d0_v, [rows, z16])
                        d1 = plsc.load_gather(d1_v, [rows, z16])
                        a = ex / (d0 + d1 + 1e-16)
                        if apb_v is not None:
                            ap0 = plsc.load_gather(apb_v, [rows, z16])
                            ap1 = plsc.load_gather(apb_v, [rows, z16 + 1])
                            a = a * (1.0 - ALPHA) + (ap0 + ap1) * (0.5 * ALPHA)
                        plsc.store_scatter(ab_v, [rows, z16], a)
                        return cy
                    lax.fori_loop(0, C // 16, grp, 0)

                pltpu.sync_copy(ab_v, a_h.at[pl.ds(off, C)])
            return carry

        lax.fori_loop(0, KPW, chunk, 0)

    return passa


def _make_pass2(heads, W, hmap):
    h2 = heads == 2
    nv = W // 16
    assert len(hmap) == nv
    ab_w = 2 if h2 else 1

    @functools.partial(
        pl.kernel,
        out_type=jax.ShapeDtypeStruct((NC, N, W), jnp.float32),
        mesh=_mesh,
        compiler_params=_sc_params,
        scratch_types=[
            pltpu.VMEM((C2,), jnp.int32), pltpu.VMEM((C2,), jnp.int32),
            pltpu.VMEM((C2,), jnp.int32), pltpu.VMEM((C2,), jnp.int32),
            pltpu.VMEM((C2, ab_w), jnp.float32), pltpu.VMEM((C2, ab_w), jnp.float32),
            pltpu.VMEM((C2, W), jnp.float32), pltpu.VMEM((C2, W), jnp.float32),
            pltpu.SemaphoreType.DMA, pltpu.SemaphoreType.DMA,
            pltpu.SemaphoreType.DMA, pltpu.SemaphoreType.DMA,
            pltpu.SemaphoreType.DMA, pltpu.SemaphoreType.DMA,
            pltpu.VMEM_SHARED((N, W), jnp.float32),
        ],
    )
    def pass2(src_h, dst_h, a_h, ftab_h, zero_h, rstp_h,
              src0, src1, dst0, dst1, ab0, ab1, fr0, fr1,
              ss0, ss1, sd0, sd1, sa0, sa1, acc_sh):
        c = lax.axis_index("c")
        s = lax.axis_index("s")
        wid = s * NC + c
        for z in range(RT // RZ):
            pltpu.sync_copy(zero_h, acc_sh.at[pl.ds(s * RT + z * RZ, RZ)])
        plsc.subcore_barrier()

        i16 = _i16()
        z16 = jnp.zeros((16,), jnp.int32)
        cols = [i16 + 16 * t for t in range(nv)]
        src_v = (src0, src1)
        dst_v = (dst0, dst1)
        ab_v = (ab0, ab1)
        fr_v = (fr0, fr1)
        sems = ((ss0, sd0, sa0), (ss1, sd1, sa1))

        def issue(b, ch):
            @pl.when(ch < NCH2)
            def _():
                off = ch * C2
                pltpu.async_copy(src_h.at[pl.ds(off, C2)], src_v[b], sems[b][0])
                pltpu.async_copy(dst_h.at[pl.ds(off, C2)], dst_v[b], sems[b][1])
                pltpu.async_copy(a_h.at[pl.ds(off, C2)], ab_v[b], sems[b][2])

        def half(b, ch):
            @pl.when(ch < NCH2)
            def _():
                off = ch * C2
                pltpu.make_async_copy(src_h.at[pl.ds(off, C2)], src_v[b],
                                      sems[b][0]).wait()
                pltpu.make_async_copy(dst_h.at[pl.ds(off, C2)], dst_v[b],
                                      sems[b][1]).wait()
                pltpu.make_async_copy(a_h.at[pl.ds(off, C2)], ab_v[b],
                                      sems[b][2]).wait()
                pltpu.sync_copy(ftab_h.at[src_v[b]], fr_v[b])
                issue(1 - b, ch + NW)

                ng = C2 // 8 if h2 else C2 // 16
                epg = 8 if h2 else 16
                frows_v = fr_v[b]
                abb_v = ab_v[b]

                def grp(g, cy):
                    for j in range(epg):
                        eloc = g * epg + j
                        rowv = z16 + eloc
                        if h2:
                            bs = {}
                            for h in set(hmap):
                                bs[h] = plsc.load_gather(abb_v, [rowv, z16 + h])
                        else:
                            bb = plsc.load_gather(abb_v, [rowv, z16])
                            bs = {h: bb for h in set(hmap)}
                        for t in range(nv):
                            r = plsc.load_gather(frows_v, [rowv, cols[t]])
                            plsc.store_scatter(frows_v, [rowv, cols[t]],
                                               r * bs[hmap[t]])
                    return cy

                lax.fori_loop(0, ng, grp, 0)
                pltpu.sync_copy(fr_v[b], acc_sh.at[dst_v[b]], add=True)

        issue(0, wid)

        def pair(kk, carry):
            half(0, wid + (2 * kk) * NW)
            half(1, wid + (2 * kk + 1) * NW)
            return carry

        lax.fori_loop(0, KPW2 // 2, pair, 0)
        plsc.subcore_barrier()
        pltpu.sync_copy(acc_sh.at[pl.ds(s * RT, RT)],
                        rstp_h.at[c, pl.ds(s * RT, RT)])

    return pass2


_pass1_h2 = _make_pass1(2)
_pass1_h1 = _make_pass1(1)
_passa_l0 = _make_passa(2, None)
_passa_l1 = _make_passa(2, 2)
_passa_l2 = _make_passa(1, 2)
_pass2_h2_00 = _make_pass2(2, 32, (0, 0))
_pass2_h2_01 = _make_pass2(2, 32, (0, 1))
_pass2_h2_11 = _make_pass2(2, 32, (1, 1))
_pass2_h1_32 = _make_pass2(1, 32, (0, 0))
_pass2_h1_16 = _make_pass2(1, 16, (0,))


@functools.partial(
    pl.kernel,
    out_type=[jax.ShapeDtypeStruct((B, DDIM), jnp.float32),
              jax.ShapeDtypeStruct((B, DDIM), jnp.float32)],
    mesh=_mesh,
    compiler_params=_sc_params,
    scratch_types=[
        pltpu.VMEM((1024,), jnp.int32),
        pltpu.VMEM((1024, DDIM), jnp.float32),
    ],
)
def _sc_decode_gather(o_h, left_h, right_h, le_h, re_h, idx_v, rows_v):
    c = lax.axis_index("c")
    s = lax.axis_index("s")
    wid = s * NC + c
    base = wid * BROW

    def chunk(k, carry):
        off = base + k * 1024
        pltpu.sync_copy(left_h.at[pl.ds(off, 1024)], idx_v)
        pltpu.sync_copy(o_h.at[idx_v], rows_v)
        pltpu.sync_copy(rows_v, le_h.at[pl.ds(off, 1024)])
        pltpu.sync_copy(right_h.at[pl.ds(off, 1024)], idx_v)
        pltpu.sync_copy(o_h.at[idx_v], rows_v)
        pltpu.sync_copy(rows_v, re_h.at[pl.ds(off, 1024)])
        return carry

    lax.fori_loop(0, BROW // 1024, chunk, 0)


BLK = 1000
NB = N // BLK


def _tmap(i):
    return (i >= N0 // BLK).astype(jnp.int32) + (i >= (N0 + N1) // BLK).astype(jnp.int32)


def _l2n(x):
    return x / jnp.maximum(jnp.sqrt(jnp.sum(x * x, axis=1, keepdims=True)), 1e-12)


def _slotmean(x):
    return (x[:, :16] + x[:, 16:32] + x[:, 32:48]) * (1.0 / 3.0)


def _elu(x):
    return jnp.where(x > 0, x, jnp.exp(jnp.minimum(x, 0.0)) - 1.0)


def _heads_el(ff, av, heads, d):
    parts = [jnp.sum(ff[:, h * d:(h + 1) * d] * av[0, h * d:(h + 1) * d][None],
                     axis=1, keepdims=True) for h in range(heads)]
    parts.append(jnp.zeros((ff.shape[0], 16 - heads), jnp.float32))
    return jnp.concatenate(parts, axis=1)


def _k0_body(f_ref, fcw_ref, fcb_ref, wb0_ref, al_ref, ar_ref, eeb_ref, aeb_ref,
             fa_ref, fb_ref, fc_ref, el_ref, er_ref, emb_ref, eet_ref):
    nt = jnp.dot(f_ref[...], fcw_ref[0], preferred_element_type=jnp.float32)
    nt = nt + fcb_ref[0, 0][None]
    ff = jnp.dot(nt, wb0_ref[0], preferred_element_type=jnp.float32)
    fa_ref[...] = ff[:, :32]
    fb_ref[...] = ff[:, 32:64]
    fc_ref[...] = ff[:, 64:96]
    el_ref[...] = _heads_el(ff, al_ref[...], 2, 48)
    er_ref[...] = _heads_el(ff, ar_ref[...], 2, 48)
    nn = jnp.maximum(jnp.sqrt(jnp.sum(nt * nt, axis=1, keepdims=True)), 1e-12)
    emb_ref[...] = nt / (3.0 * nn)
    eet_ref[...] = jnp.dot(eeb_ref[...], aeb_ref[...],
                           preferred_element_type=jnp.float32)


def _k12_body(pa_ref, pb_ref, pc_ref, rv_ref, bias_ref, wb_ref, rwb_ref,
              al_ref, ar_ref,
              emb_ref, fa_ref, fb_ref, fc_ref, el_ref, er_ref, rvo_ref,
              *, heads_out, d_out):
    rst = jnp.concatenate([pa_ref[0] + pa_ref[1], pb_ref[0] + pb_ref[1],
                           pc_ref[0] + pc_ref[1]], axis=1)
    if rv_ref is not None:
        rst = rst + rv_ref[...]
    rst = rst + bias_ref[...]
    h = _elu(rst)
    hm = (h[:, :48] + h[:, 48:]) * 0.5
    emb_ref[...] = _slotmean(_l2n(hm))
    ff = jnp.dot(h, wb_ref[...], preferred_element_type=jnp.float32)
    rvo_ref[...] = jnp.dot(h, rwb_ref[...], preferred_element_type=jnp.float32)
    fa_ref[...] = ff[:, :32]
    if d_out == 96:
        fb_ref[...] = ff[:, 32:64]
        fc_ref[...] = ff[:, 64:96]
    else:
        fb_ref[...] = ff[:, 32:48]
    el_ref[...] = _heads_el(ff, al_ref[...], heads_out, d_out // heads_out)
    er_ref[...] = _heads_el(ff, ar_ref[...], heads_out, d_out // heads_out)


def _k3_body(pa_ref, pb_ref, rv_ref, bias_ref, e0_ref, e1_ref, e2_ref, o_ref):
    logits = jnp.concatenate([pa_ref[0] + pa_ref[1], pb_ref[0] + pb_ref[1]],
                             axis=1)
    logits = logits + rv_ref[...] + bias_ref[...]
    emb3 = _slotmean(_l2n(logits))
    o_ref[...] = jnp.concatenate([e0_ref[...], e1_ref[...], e2_ref[...], emb3],
                                 axis=1)


def _decode_body(le_ref, re_ref, mid_ref, w_ref, out_ref):
    le = le_ref[...]
    re = re_ref[...]
    mid = mid_ref[...]
    acc = jnp.zeros((le.shape[0], 1), jnp.float32)
    for r in range(NUM_ETYPES):
        t = jnp.dot(le, w_ref[r], preferred_element_type=jnp.float32)
        rs = jnp.sum(t * re, axis=1, keepdims=True)
        acc = acc + jnp.where(mid == r, rs, 0.0)
    out_ref[...] = acc


def _rowspec(w):
    return pl.BlockSpec((BLK, w), lambda i: (i, 0))


def _pspec(w):
    return pl.BlockSpec((2, BLK, w), lambda i: (0, i, 0))


def _whole(shape):
    nd = len(shape)
    return pl.BlockSpec(shape, lambda i: (0,) * nd)


def kernel(feat0, feat1, feat2, edge_index, e_feat, left, right, mid, params):
    p = params
    src = edge_index[0].astype(jnp.int32)
    dst = edge_index[1].astype(jnp.int32)
    ef = e_feat.astype(jnp.int32)

    featcat = jnp.concatenate([feat0, feat1, feat2], axis=0)
    fcw = jnp.stack([p['fc_w0'], p['fc_w1'], p['fc_w2']])
    fcb = jnp.stack([p['fc_b0'], p['fc_b1'], p['fc_b2']]).reshape(3, 1, 16)

    def blockdiag(W, heads, out):
        Wr = W.reshape(NT, W.shape[1], heads, out)
        Wb = jnp.zeros((NT * W.shape[1], heads * NT * out), jnp.float32)
        for t in range(NT):
            for h in range(heads):
                Wb = Wb.at[t * W.shape[1]:(t + 1) * W.shape[1],
                           h * NT * out + t * out:h * NT * out + (t + 1) * out
                           ].set(Wr[t, :, h, :])
        return Wb

    wb0s = blockdiag(p['W0'], 2, 16).reshape(NT, 16, 96)
    wb1 = blockdiag(p['W1'], 2, 16)
    rwb1 = blockdiag(p['res_W1'], 2, 16)
    wb2 = blockdiag(p['W2'], 1, 16)
    rwb2 = blockdiag(p['res_W2'], 1, 16)
    al = [p[f'attn_l{l}'].reshape(1, -1) for l in range(3)]
    ar = [p[f'attn_r{l}'].reshape(1, -1) for l in range(3)]
    bias = [p[f'bias{l}'].reshape(1, -1) for l in range(3)]
    eeb = jnp.zeros((16, 48), jnp.float32)
    aeb = jnp.zeros((48, 128), jnp.float32)
    for l in range(3):
        eeb = eeb.at[5 * l:5 * l + 5, 16 * l:16 * l + 16].set(p[f'edge_emb{l}'])
        aeb = aeb.at[16 * l:16 * l + 16, 2 * l:2 * l + HEADS[l]].set(p[f'attn_e{l}'])
    zden = jnp.zeros((RZ, 16), jnp.float32)
    zex = jnp.zeros((C, 16), jnp.float32)
    zero32 = jnp.zeros((RZ, 32), jnp.float32)
    zero16 = jnp.zeros((RZ, 16), jnp.float32)

    wmap = lambda i: (_tmap(i), 0, 0)
    fA0, fB0, fC0, el0, er0, emb0, eetabs = pl.pallas_call(
        _k0_body,
        grid=(NB,),
        in_specs=[
            _rowspec(128),
            pl.BlockSpec((1, 128, 16), wmap),
            pl.BlockSpec((1, 1, 16), wmap),
            pl.BlockSpec((1, 16, 96), wmap),
            _whole((1, 96)), _whole((1, 96)),
            _whole((16, 48)), _whole((48, 128)),
        ],
        out_specs=[_rowspec(32), _rowspec(32), _rowspec(32),
                   _rowspec(16), _rowspec(16), _rowspec(16),
                   _whole((16, 128))],
        out_shape=[jax.ShapeDtypeStruct((N, 32), jnp.float32),
                   jax.ShapeDtypeStruct((N, 32), jnp.float32),
                   jax.ShapeDtypeStruct((N, 32), jnp.float32),
                   jax.ShapeDtypeStruct((N, 16), jnp.float32),
                   jax.ShapeDtypeStruct((N, 16), jnp.float32),
                   jax.ShapeDtypeStruct((N, 16), jnp.float32),
                   jax.ShapeDtypeStruct((16, 128), jnp.float32)],
    )(featcat, fcw, fcb, wb0s, al[0], ar[0], eeb, aeb)

    def eet_flat(l):
        t = eetabs[5 * l:5 * l + 5, 2 * l:2 * l + HEADS[l]]
        return jnp.zeros((16,), jnp.float32).at[:5 * HEADS[l]].set(t.reshape(-1))

    ex0, denp0 = _pass1_h2(src, dst, ef, el0, er0, eet_flat(0), zden, zex)
    a0 = _passa_l0(dst, ex0, denp0[0], denp0[1])
    pA0 = _pass2_h2_00(src, dst, a0, fA0, zero32)
    pB0 = _pass2_h2_01(src, dst, a0, fB0, zero32)
    pC0 = _pass2_h2_11(src, dst, a0, fC0, zero32)

    k1 = functools.partial(_k12_body, heads_out=2, d_out=96)

    def k1_body(pa, pb, pc, bias_r, wb_r, rwb_r, al_r, ar_r,
                emb_r, fa_r, fb_r, fc_r, el_r, er_r, rvo_r):
        k1(pa, pb, pc, None, bias_r, wb_r, rwb_r, al_r, ar_r,
           emb_r, fa_r, fb_r, fc_r, el_r, er_r, rvo_r)

    emb1, fA1, fB1, fC1, el1, er1, rv1 = pl.pallas_call(
        k1_body,
        grid=(NB,),
        in_specs=[_pspec(32), _pspec(32), _pspec(32),
                  _whole((1, 96)), _whole((96, 96)), _whole((96, 96)),
                  _whole((1, 96)), _whole((1, 96))],
        out_specs=[_rowspec(16), _rowspec(32), _rowspec(32), _rowspec(32),
                   _rowspec(16), _rowspec(16), _rowspec(96)],
        out_shape=[jax.ShapeDtypeStruct((N, 16), jnp.float32),
                   jax.ShapeDtypeStruct((N, 32), jnp.float32),
                   jax.ShapeDtypeStruct((N, 32), jnp.float32),
                   jax.ShapeDtypeStruct((N, 32), jnp.float32),
                   jax.ShapeDtypeStruct((N, 16), jnp.float32),
                   jax.ShapeDtypeStruct((N, 16), jnp.float32),
                   jax.ShapeDtypeStruct((N, 96), jnp.float32)],
    )(pA0, pB0, pC0, bias[0], wb1, rwb1, al[1], ar[1])

    ex1, denp1 = _pass1_h2(src, dst, ef, el1, er1, eet_flat(1), zden, zex)
    a1 = _passa_l1(dst, ex1, denp1[0], denp1[1], a0)
    pA1 = _pass2_h2_00(src, dst, a1, fA1, zero32)
    pB1 = _pass2_h2_01(src, dst, a1, fB1, zero32)
    pC1 = _pass2_h2_11(src, dst, a1, fC1, zero32)

    k2 = functools.partial(_k12_body, heads_out=1, d_out=48)

    def k2_body(pa, pb, pc, rv_r, bias_r, wb_r, rwb_r, al_r, ar_r,
                emb_r, fa_r, fb_r, el_r, er_r, rvo_r):
        k2(pa, pb, pc, rv_r, bias_r, wb_r, rwb_r, al_r, ar_r,
           emb_r, fa_r, fb_r, None, el_r, er_r, rvo_r)

    emb2, fA2, fB2, el2, er2, rv2 = pl.pallas_call(
        k2_body,
        grid=(NB,),
        in_specs=[_pspec(32), _pspec(32), _pspec(32), _rowspec(96),
                  _whole((1, 96)), _whole((96, 48)), _whole((96, 48)),
                  _whole((1, 48)), _whole((1, 48))],
        out_specs=[_rowspec(16), _rowspec(32), _rowspec(16),
                   _rowspec(16), _rowspec(16), _rowspec(48)],
        out_shape=[jax.ShapeDtypeStruct((N, 16), jnp.float32),
                   jax.ShapeDtypeStruct((N, 32), jnp.float32),
                   jax.ShapeDtypeStruct((N, 16), jnp.float32),
                   jax.ShapeDtypeStruct((N, 16), jnp.float32),
                   jax.ShapeDtypeStruct((N, 16), jnp.float32),
                   jax.ShapeDtypeStruct((N, 48), jnp.float32)],
    )(pA1, pB1, pC1, rv1, bias[1], wb2, rwb2, al[2], ar[2])

    ex2, denp2 = _pass1_h1(src, dst, ef, el2, er2, eet_flat(2), zden, zex)
    a2 = _passa_l2(dst, ex2, denp2[0], denp2[1], a1)
    pA2 = _pass2_h1_32(src, dst, a2, fA2, zero32)
    pB2 = _pass2_h1_16(src, dst, a2, fB2, zero16)

    o = pl.pallas_call(
        _k3_body,
        grid=(NB,),
        in_specs=[_pspec(32), _pspec(16), _rowspec(48), _whole((1, 48)),
                  _rowspec(16), _rowspec(16), _rowspec(16)],
        out_specs=_rowspec(64),
        out_shape=jax.ShapeDtypeStruct((N, 64), jnp.float32),
    )(pA2, pB2, rv2, bias[2], emb0, emb1, emb2)

    le, re = _sc_decode_gather(o, left.astype(jnp.int32), right.astype(jnp.int32))
    blk = 2048
    out = pl.pallas_call(
        _decode_body,
        grid=(B // blk,),
        in_specs=[
            pl.BlockSpec((blk, DDIM), lambda i: (i, 0)),
            pl.BlockSpec((blk, DDIM), lambda i: (i, 0)),
            pl.BlockSpec((blk, 1), lambda i: (i, 0)),
            pl.BlockSpec((NUM_ETYPES, DDIM, DDIM), lambda i: (0, 0, 0)),
        ],
        out_specs=pl.BlockSpec((blk, 1), lambda i: (i, 0)),
        out_shape=jax.ShapeDtypeStruct((B, 1), jnp.float32),
    )(le, re, mid.reshape(B, 1).astype(jnp.int32), p['dist_W'])
    return out[:, 0]
